# baseline (speedup 1.0000x reference)
import functools

import numpy as np
import jax
import jax.numpy as jnp
from jax import lax
from jax.experimental import pallas as pl
from jax.experimental.pallas import tpu as pltpu
from jax.experimental.pallas import tpu_sc as plsc

_NC = 2
_NS = 16
_NW = _NC * _NS


@functools.lru_cache(maxsize=None)
def _pe_const(seq_len: int, d_model: int):
    pos = np.arange(seq_len)[:, None].astype(np.float64)
    i = np.arange(d_model)[None, :]
    angle_rates = 1.0 / np.power(10000.0, (2 * (i // 2)) / float(d_model))
    angles = pos * angle_rates
    pe = np.zeros((seq_len, d_model), dtype=np.float32)
    pe[:, 0::2] = np.sin(angles[:, 0::2])
    pe[:, 1::2] = np.cos(angles[:, 1::2])
    return jnp.asarray(pe)


@functools.lru_cache(maxsize=None)
def _build(B: int, S: int, D: int):
    SPW = S // _NW
    NV = D // 16

    mesh = plsc.VectorSubcoreMesh(core_axis_name="c", subcore_axis_name="s")

    @functools.partial(
        pl.kernel,
        out_type=jax.ShapeDtypeStruct((B, S, D), jnp.float32),
        mesh=mesh,
        scratch_types=[
            pltpu.VMEM((SPW, D), jnp.float32),
            pltpu.VMEM((SPW, D), jnp.float32),
            pltpu.VMEM((SPW,), jnp.int32),
            pltpu.VMEM((SPW + 16,), jnp.int32),
            pltpu.VMEM((3 * D,), jnp.float32),
            pltpu.SemaphoreType.DMA,
        ],
    )
    def emb(seq_hbm, lab_hbm, tok_hbm, segf_hbm, pe_hbm, out_hbm,
            pe_v, rows_v, idx_v, lab_v, seg_v, sem):
        wid = lax.axis_index("s") * _NC + lax.axis_index("c")
        s0 = wid * SPW
        pltpu.sync_copy(pe_hbm.at[pl.ds(s0, SPW)], pe_v)
        pltpu.sync_copy(segf_hbm, seg_v)
        for b in range(B):
            pltpu.sync_copy(seq_hbm.at[b, pl.ds(s0, SPW)], idx_v)
            pltpu.sync_copy(lab_hbm.at[b, pl.ds(s0, SPW)], lab_v.at[pl.ds(0, SPW)])
            pltpu.async_copy(tok_hbm.at[idx_v], rows_v, sem).wait()

            def tok_body(i, carry):
                base = lab_v[pl.ds(i, 16)][0] * D
                for j in range(NV):
                    sl = pl.ds(j * 16, 16)
                    segvals = seg_v[pl.ds(base + j * 16, 16)]
                    rows_v[i, sl] = rows_v[i, sl] + pe_v[i, sl] + segvals
                return carry

            lax.fori_loop(0, SPW, tok_body, 0)
            pltpu.sync_copy(rows_v, out_hbm.at[b, pl.ds(s0, SPW)])

    return emb


def kernel(sequence, segment_label, token_table, segment_table):
    B, S = sequence.shape
    D = token_table.shape[1]
    pe = _pe_const(S, D)
    seq = sequence.astype(jnp.int32)
    lab = segment_label.astype(jnp.int32)
    segf = segment_table.astype(jnp.float32).reshape(-1)
    return _build(B, S, D)(seq, lab, token_table.astype(jnp.float32), segf, pe)


# speedup vs baseline: 1.0021x; 1.0021x over previous
import functools

import numpy as np
import jax
import jax.numpy as jnp
from jax import lax
from jax.experimental import pallas as pl
from jax.experimental.pallas import tpu as pltpu
from jax.experimental.pallas import tpu_sc as plsc

_NC = 2
_NS = 16
_NW = _NC * _NS


@functools.lru_cache(maxsize=None)
def _pe_const(seq_len: int, d_model: int):
    pos = np.arange(seq_len)[:, None].astype(np.float64)
    i = np.arange(d_model)[None, :]
    angle_rates = 1.0 / np.power(10000.0, (2 * (i // 2)) / float(d_model))
    angles = pos * angle_rates
    pe = np.zeros((seq_len, d_model), dtype=np.float32)
    pe[:, 0::2] = np.sin(angles[:, 0::2])
    pe[:, 1::2] = np.cos(angles[:, 1::2])
    return jnp.asarray(pe)


@functools.lru_cache(maxsize=None)
def _build(B: int, S: int, D: int):
    SPW = S // _NW
    NV = D // 16
    CH = 32
    NCH = B * SPW // CH
    HPB = SPW // CH

    mesh = plsc.VectorSubcoreMesh(core_axis_name="c", subcore_axis_name="s")

    @functools.partial(
        pl.kernel,
        out_type=jax.ShapeDtypeStruct((B, S, D), jnp.float32),
        mesh=mesh,
        scratch_types=[
            pltpu.VMEM((SPW, D), jnp.float32),
            pltpu.VMEM((CH, D), jnp.float32),
            pltpu.VMEM((CH, D), jnp.float32),
            pltpu.VMEM((B * SPW,), jnp.int32),
            pltpu.VMEM((B * SPW + 16,), jnp.int32),
            pltpu.VMEM((3 * D,), jnp.float32),
            pltpu.SemaphoreType.DMA,
            pltpu.SemaphoreType.DMA,
            pltpu.SemaphoreType.DMA,
            pltpu.SemaphoreType.DMA,
        ],
    )
    def emb(seq_hbm, lab_hbm, tok_hbm, segf_hbm, pe_hbm, out_hbm,
            pe_v, rows0, rows1, idx_v, lab_v, seg_v,
            gsem0, gsem1, wsem0, wsem1):
        wid = lax.axis_index("s") * _NC + lax.axis_index("c")
        s0 = wid * SPW
        pltpu.sync_copy(pe_hbm.at[pl.ds(s0, SPW)], pe_v)
        pltpu.sync_copy(segf_hbm, seg_v)
        for b in range(B):
            pltpu.sync_copy(seq_hbm.at[b, pl.ds(s0, SPW)],
                            idx_v.at[pl.ds(b * SPW, SPW)])
            pltpu.sync_copy(lab_hbm.at[b, pl.ds(s0, SPW)],
                            lab_v.at[pl.ds(b * SPW, SPW)])

        rows = (rows0, rows1)
        gsem = (gsem0, gsem1)
        wsem = (wsem0, wsem1)
        gd = [None] * NCH
        wd = [None] * NCH

        def start_gather(c):
            gd[c] = pltpu.async_copy(
                tok_hbm.at[idx_v.at[pl.ds(c * CH, CH)]], rows[c % 2], gsem[c % 2])

        def compute(c):
            b, h = divmod(c, HPB)
            buf = rows[c % 2]

            def tok_body(i, carry):
                base = lab_v[pl.ds(c * CH + i, 16)][0] * D
                prow = h * CH + i
                for j in range(NV):
                    sl = pl.ds(j * 16, 16)
                    buf[i, sl] = (buf[i, sl] + pe_v[prow, sl]
                                  + seg_v[pl.ds(base + j * 16, 16)])
                return carry

            lax.fori_loop(0, CH, tok_body, 0)

        def start_write(c):
            b, h = divmod(c, HPB)
            wd[c] = pltpu.async_copy(
                rows[c % 2], out_hbm.at[b, pl.ds(s0 + h * CH, CH)], wsem[c % 2])

        start_gather(0)
        for c in range(NCH):
            if c + 1 < NCH:
                if c - 1 >= 0:
                    wd[c - 1].wait()
                start_gather(c + 1)
            gd[c].wait()
            compute(c)
            start_write(c)
        wd[NCH - 2].wait()
        wd[NCH - 1].wait()

    return emb


def kernel(sequence, segment_label, token_table, segment_table):
    B, S = sequence.shape
    D = token_table.shape[1]
    pe = _pe_const(S, D)
    seq = sequence.astype(jnp.int32)
    lab = segment_label.astype(jnp.int32)
    segf = segment_table.astype(jnp.float32).reshape(-1)
    return _build(B, S, D)(seq, lab, token_table.astype(jnp.float32), segf, pe)
